# baseline (speedup 1.0000x reference)
import functools

import jax
import jax.numpy as jnp
from jax.experimental import pallas as pl
from jax.experimental.pallas import tpu as pltpu

_N_BINS = 16
_TAIL = 3.0
_MIN_VAL = 1e-3
_MIN_TOTAL = _N_BINS * _MIN_VAL
_WMIN = 2.0 * _TAIL * _MIN_VAL
_WSCALE = 2.0 * _TAIL * (1.0 - _MIN_TOTAL)
_NP = 3 * _N_BINS - 1

_G = 4
_BR = 256
_BC = 1024

_CORE = pltpu.GridDimensionSemantics.CORE_PARALLEL
_ARB = pltpu.GridDimensionSemantics.ARBITRARY


def _mlp_kernel(x1_ref, x2_ref, w0t_ref, w1t_ref, b0_ref, b1_ref,
                out_ref, x2t_ref):
    x1t = x1_ref[...].astype(jnp.bfloat16).T
    x2t_ref[...] = x2_ref[...].T
    h0 = jnp.dot(w0t_ref[...], x1t, preferred_element_type=jnp.float32)
    h0 = jnp.maximum(h0 + b0_ref[...], 0.0).astype(jnp.bfloat16)
    h1 = jnp.dot(w1t_ref[...], h0, preferred_element_type=jnp.float32)
    out_ref[...] = jnp.maximum(h1 + b1_ref[...], 0.0).astype(jnp.bfloat16)


def _unt_kernel(yt_ref, y_ref):
    y_ref[...] = yt_ref[...].T


def _softplus(v):
    return jnp.log(1.0 + jnp.exp(v))


_N_STAGES = 7
_CHUNK_PLANES = (7, 7, 7, 7, 7, 6, 6)


def _spline_stages(cg, p_ref, x):
    def plane(j):
        return p_ref[j * cg:(j + 1) * cg, :]

    xc = jnp.clip(x, -_TAIL, _TAIL)

    ew = [jnp.exp(plane(j)) for j in range(_N_BINS)]
    sw = functools.reduce(jnp.add, ew)
    scale_w = _WSCALE / sw
    yield None

    eh = [jnp.exp(plane(_N_BINS + j)) for j in range(_N_BINS)]
    sh = functools.reduce(jnp.add, eh)
    scale_h = _WSCALE / sh
    yield None

    zeros = jnp.zeros_like(x)
    acc_x = zeros
    acc_y = zeros
    acc_w = zeros
    acc_h = zeros
    acc_d0 = zeros
    acc_d1 = zeros
    cw = None
    ch = None
    dprev = None
    m_prev = None
    for k in range(_N_BINS):
        wb = _WMIN + ew[k] * scale_w
        hb = _WMIN + eh[k] * scale_h
        cw1 = (cw + wb) if cw is not None else (wb - _TAIL)
        ch1 = (ch + hb) if ch is not None else (hb - _TAIL)
        if k == 0:
            sel = xc < cw1
            m_prev = sel
        elif k < _N_BINS - 1:
            m1 = xc < cw1
            sel = m1 & ~m_prev
            m_prev = m1
        else:
            sel = ~m_prev
        if k < _N_BINS - 1:
            dnext = _MIN_VAL + _softplus(plane(2 * _N_BINS + k))
        else:
            dnext = None
        acc_x = jnp.where(sel, cw if cw is not None else -_TAIL, acc_x)
        acc_y = jnp.where(sel, ch if ch is not None else -_TAIL, acc_y)
        acc_w = jnp.where(sel, wb, acc_w)
        acc_h = jnp.where(sel, hb, acc_h)
        acc_d0 = jnp.where(sel, dprev if dprev is not None else 1.0, acc_d0)
        acc_d1 = jnp.where(sel, dnext if dnext is not None else 1.0, acc_d1)
        cw, ch, dprev = cw1, ch1, dnext
        if k in (3, 7, 11, 15):
            yield None

    rw = 1.0 / acc_w
    sk = acc_h * rw
    theta = (xc - acc_x) * rw
    omt = 1.0 - theta
    t1m = theta * omt
    th2 = theta * theta
    denom = sk + (acc_d0 + acc_d1 - 2.0 * sk) * t1m
    rden = 1.0 / denom
    y = acc_y + acc_h * (sk * th2 + acc_d0 * t1m) * rden
    deriv = (sk * sk) * (acc_d1 * th2 + 2.0 * sk * t1m + acc_d0 * omt * omt)
    deriv = deriv * (rden * rden)
    inside = (x > -_TAIL) & (x < _TAIL)
    yout = jnp.where(inside, y, x)
    ldt = jnp.where(inside, jnp.log(deriv), 0.0)
    yield yout, jnp.sum(ldt, axis=0, keepdims=True)


def _spline_kernel(cg, x2t_ref, hdnt_ref, w2t_ref, b2_ref, yt_ref, ld_ref,
                   pa_ref, pb_ref):
    def dot_chunk(p_ref, cols, off, sz):
        h = hdnt_ref[:, cols]
        p = jnp.dot(w2t_ref[off:off + sz, :], h,
                    preferred_element_type=jnp.float32)
        p_ref[off:off + sz, :] = p + b2_ref[off:off + sz, :]

    cols_a = slice(0, _BR)
    cols_b = slice(_BR, 2 * _BR)

    off = 0
    for npl in _CHUNK_PLANES:
        dot_chunk(pa_ref, cols_a, off * cg, npl * cg)
        off += npl

    gen_a = _spline_stages(cg, pa_ref, x2t_ref[:, cols_a])
    off = 0
    res_a = None
    for npl in _CHUNK_PLANES:
        dot_chunk(pb_ref, cols_b, off * cg, npl * cg)
        off += npl
        res_a = next(gen_a)
    y_a, ld_a = res_a
    yt_ref[:, cols_a] = y_a
    ld_ref[:, :, cols_a] = ld_a[None]

    gen_b = _spline_stages(cg, pb_ref, x2t_ref[:, cols_b])
    res_b = None
    for _ in range(_N_STAGES):
        res_b = next(gen_b)
    y_b, ld_b = res_b
    yt_ref[:, cols_b] = y_b
    ld_ref[:, :, cols_b] = ld_b[None]


def kernel(x1, x2, W0, b0, W1, b1, W2, b2):
    B, D1 = x1.shape
    D2 = x2.shape[1]
    DFF = W0.shape[1]
    cg = D2 // _G

    w0t = W0.T.astype(jnp.bfloat16)
    w1t = W1.T.astype(jnp.bfloat16)
    b0r = jnp.broadcast_to(b0[:, None], (DFF, _BC))
    b1r = jnp.broadcast_to(b1[:, None], (DFF, _BC))

    hdnt, x2t = pl.pallas_call(
        _mlp_kernel,
        grid=(B // _BC,),
        in_specs=[
            pl.BlockSpec((_BC, D1), lambda c: (c, 0)),
            pl.BlockSpec((_BC, D2), lambda c: (c, 0)),
            pl.BlockSpec((DFF, D1), lambda c: (0, 0)),
            pl.BlockSpec((DFF, DFF), lambda c: (0, 0)),
            pl.BlockSpec((DFF, _BC), lambda c: (0, 0)),
            pl.BlockSpec((DFF, _BC), lambda c: (0, 0)),
        ],
        out_specs=[
            pl.BlockSpec((DFF, _BC), lambda c: (0, c)),
            pl.BlockSpec((D2, _BC), lambda c: (0, c)),
        ],
        out_shape=[
            jax.ShapeDtypeStruct((DFF, B), jnp.bfloat16),
            jax.ShapeDtypeStruct((D2, B), jnp.float32),
        ],
        compiler_params=pltpu.CompilerParams(
            dimension_semantics=(_ARB,),
            vmem_limit_bytes=100 * 1024 * 1024,
        ),
    )(x1, x2, w0t, w1t, b0r, b1r)

    w2tp = (W2.reshape(DFF, _G, cg, _NP).transpose(1, 3, 2, 0)
            .reshape(_G * _NP * cg, DFF).astype(jnp.bfloat16))
    b2p = b2.reshape(_G, cg, _NP).transpose(0, 2, 1).reshape(_G * _NP * cg)
    b2r = jnp.broadcast_to(b2p[:, None], (_G * _NP * cg, _BR))

    yt, ldp = pl.pallas_call(
        functools.partial(_spline_kernel, cg),
        grid=(_G, B // (2 * _BR)),
        in_specs=[
            pl.BlockSpec((cg, 2 * _BR), lambda g, r: (g, r)),
            pl.BlockSpec((DFF, 2 * _BR), lambda g, r: (0, r)),
            pl.BlockSpec((_NP * cg, DFF), lambda g, r: (g, 0)),
            pl.BlockSpec((_NP * cg, _BR), lambda g, r: (g, 0)),
        ],
        out_specs=[
            pl.BlockSpec((cg, 2 * _BR), lambda g, r: (g, r)),
            pl.BlockSpec((1, 1, 2 * _BR), lambda g, r: (g, 0, r)),
        ],
        out_shape=[
            jax.ShapeDtypeStruct((D2, B), jnp.float32),
            jax.ShapeDtypeStruct((_G, 1, B), jnp.float32),
        ],
        scratch_shapes=[
            pltpu.VMEM((_NP * cg, _BR), jnp.float32),
            pltpu.VMEM((_NP * cg, _BR), jnp.float32),
        ],
        compiler_params=pltpu.CompilerParams(
            dimension_semantics=(_ARB, _ARB),
            vmem_limit_bytes=100 * 1024 * 1024,
        ),
    )(x2t, hdnt, w2tp, b2r)

    y = pl.pallas_call(
        _unt_kernel,
        grid=(B // _BC,),
        in_specs=[pl.BlockSpec((D2, _BC), lambda c: (0, c))],
        out_specs=pl.BlockSpec((_BC, D2), lambda c: (c, 0)),
        out_shape=jax.ShapeDtypeStruct((B, D2), jnp.float32),
        compiler_params=pltpu.CompilerParams(
            dimension_semantics=(_ARB,),
            vmem_limit_bytes=100 * 1024 * 1024,
        ),
    )(yt)

    return y, ldp.sum(axis=(0, 1))


# speedup vs baseline: 1.0094x; 1.0094x over previous
import functools

import jax
import jax.numpy as jnp
from jax.experimental import pallas as pl
from jax.experimental.pallas import tpu as pltpu

_N_BINS = 16
_TAIL = 3.0
_MIN_VAL = 1e-3
_MIN_TOTAL = _N_BINS * _MIN_VAL
_WMIN = 2.0 * _TAIL * _MIN_VAL
_WSCALE = 2.0 * _TAIL * (1.0 - _MIN_TOTAL)
_NP = 3 * _N_BINS - 1

_G = 4
_BR = 256
_BC = 1024

_ARB = pltpu.GridDimensionSemantics.ARBITRARY


def _mlp_kernel(x1_ref, x2_ref, w0t_ref, w1t_ref, b0_ref, b1_ref,
                out_ref, x2t_ref):
    x1t = x1_ref[...].astype(jnp.bfloat16).T
    x2t_ref[...] = x2_ref[...].T
    h0 = jnp.dot(w0t_ref[...], x1t, preferred_element_type=jnp.float32)
    h0 = jnp.maximum(h0 + b0_ref[...], 0.0).astype(jnp.bfloat16)
    h1 = jnp.dot(w1t_ref[...], h0, preferred_element_type=jnp.float32)
    out_ref[...] = jnp.maximum(h1 + b1_ref[...], 0.0).astype(jnp.bfloat16)


def _softplus(v):
    return jnp.log(1.0 + jnp.exp(v))


_N_STAGES = 7
_CHUNK_PLANES = (7, 7, 7, 7, 7, 6, 6)


def _spline_stages(cg, p_ref, x):
    def plane(j):
        return p_ref[j * cg:(j + 1) * cg, :]

    xc = jnp.clip(x, -_TAIL, _TAIL)

    ew = [jnp.exp(plane(j)) for j in range(_N_BINS)]
    sw = functools.reduce(jnp.add, ew)
    scale_w = _WSCALE / sw
    yield None

    eh = [jnp.exp(plane(_N_BINS + j)) for j in range(_N_BINS)]
    sh = functools.reduce(jnp.add, eh)
    scale_h = _WSCALE / sh
    yield None

    zeros = jnp.zeros_like(x)
    acc_x = zeros
    acc_y = zeros
    acc_w = zeros
    acc_h = zeros
    acc_d0 = zeros
    acc_d1 = zeros
    cw = None
    ch = None
    dprev = None
    m_prev = None
    for k in range(_N_BINS):
        wb = _WMIN + ew[k] * scale_w
        hb = _WMIN + eh[k] * scale_h
        cw1 = (cw + wb) if cw is not None else (wb - _TAIL)
        ch1 = (ch + hb) if ch is not None else (hb - _TAIL)
        if k == 0:
            sel = xc < cw1
            m_prev = sel
        elif k < _N_BINS - 1:
            m1 = xc < cw1
            sel = m1 & ~m_prev
            m_prev = m1
        else:
            sel = ~m_prev
        if k < _N_BINS - 1:
            dnext = _MIN_VAL + _softplus(plane(2 * _N_BINS + k))
        else:
            dnext = None
        acc_x = jnp.where(sel, cw if cw is not None else -_TAIL, acc_x)
        acc_y = jnp.where(sel, ch if ch is not None else -_TAIL, acc_y)
        acc_w = jnp.where(sel, wb, acc_w)
        acc_h = jnp.where(sel, hb, acc_h)
        acc_d0 = jnp.where(sel, dprev if dprev is not None else 1.0, acc_d0)
        acc_d1 = jnp.where(sel, dnext if dnext is not None else 1.0, acc_d1)
        cw, ch, dprev = cw1, ch1, dnext
        if k in (3, 7, 11, 15):
            yield None

    rw = 1.0 / acc_w
    sk = acc_h * rw
    theta = (xc - acc_x) * rw
    omt = 1.0 - theta
    t1m = theta * omt
    th2 = theta * theta
    denom = sk + (acc_d0 + acc_d1 - 2.0 * sk) * t1m
    rden = 1.0 / denom
    y = acc_y + acc_h * (sk * th2 + acc_d0 * t1m) * rden
    deriv = (sk * sk) * (acc_d1 * th2 + 2.0 * sk * t1m + acc_d0 * omt * omt)
    deriv = deriv * (rden * rden)
    inside = (x > -_TAIL) & (x < _TAIL)
    yout = jnp.where(inside, y, x)
    ldt = jnp.where(inside, jnp.log(deriv), 0.0)
    yield yout, jnp.sum(ldt, axis=0, keepdims=True)


def _spline_kernel(cg, x2t_ref, hdnt_ref, w2t_ref, b2_ref, yt_ref, ld_ref,
                   pa_ref, pb_ref):
    def dot_chunk(p_ref, cols, off, sz):
        h = hdnt_ref[:, cols]
        p = jnp.dot(w2t_ref[off:off + sz, :], h,
                    preferred_element_type=jnp.float32)
        p_ref[off:off + sz, :] = p + b2_ref[off:off + sz, :]

    cols_a = slice(0, _BR)
    cols_b = slice(_BR, 2 * _BR)

    off = 0
    for npl in _CHUNK_PLANES:
        dot_chunk(pa_ref, cols_a, off * cg, npl * cg)
        off += npl

    gen_a = _spline_stages(cg, pa_ref, x2t_ref[:, cols_a])
    off = 0
    res_a = None
    for npl in _CHUNK_PLANES:
        dot_chunk(pb_ref, cols_b, off * cg, npl * cg)
        off += npl
        res_a = next(gen_a)
    y_a, ld_a = res_a
    yt_ref[:, cols_a] = y_a
    ld_ref[:, :, cols_a] = ld_a[None]

    gen_b = _spline_stages(cg, pb_ref, x2t_ref[:, cols_b])
    res_b = None
    for _ in range(_N_STAGES):
        res_b = next(gen_b)
    y_b, ld_b = res_b
    yt_ref[:, cols_b] = y_b
    ld_ref[:, :, cols_b] = ld_b[None]


def kernel(x1, x2, W0, b0, W1, b1, W2, b2):
    B, D1 = x1.shape
    D2 = x2.shape[1]
    DFF = W0.shape[1]
    cg = D2 // _G

    w0t = W0.T.astype(jnp.bfloat16)
    w1t = W1.T.astype(jnp.bfloat16)
    b0r = jnp.broadcast_to(b0[:, None], (DFF, _BC))
    b1r = jnp.broadcast_to(b1[:, None], (DFF, _BC))

    hdnt, x2t = pl.pallas_call(
        _mlp_kernel,
        grid=(B // _BC,),
        in_specs=[
            pl.BlockSpec((_BC, D1), lambda c: (c, 0)),
            pl.BlockSpec((_BC, D2), lambda c: (c, 0)),
            pl.BlockSpec((DFF, D1), lambda c: (0, 0)),
            pl.BlockSpec((DFF, DFF), lambda c: (0, 0)),
            pl.BlockSpec((DFF, _BC), lambda c: (0, 0)),
            pl.BlockSpec((DFF, _BC), lambda c: (0, 0)),
        ],
        out_specs=[
            pl.BlockSpec((DFF, _BC), lambda c: (0, c)),
            pl.BlockSpec((D2, _BC), lambda c: (0, c)),
        ],
        out_shape=[
            jax.ShapeDtypeStruct((DFF, B), jnp.bfloat16),
            jax.ShapeDtypeStruct((D2, B), jnp.float32),
        ],
        compiler_params=pltpu.CompilerParams(
            dimension_semantics=(_ARB,),
            vmem_limit_bytes=100 * 1024 * 1024,
        ),
    )(x1, x2, w0t, w1t, b0r, b1r)

    w2tp = (W2.reshape(DFF, _G, cg, _NP).transpose(1, 3, 2, 0)
            .reshape(_G * _NP * cg, DFF).astype(jnp.bfloat16))
    b2p = b2.reshape(_G, cg, _NP).transpose(0, 2, 1).reshape(_G * _NP * cg)
    b2r = jnp.broadcast_to(b2p[:, None], (_G * _NP * cg, _BR))

    yt, ldp = pl.pallas_call(
        functools.partial(_spline_kernel, cg),
        grid=(_G, B // (2 * _BR)),
        in_specs=[
            pl.BlockSpec((cg, 2 * _BR), lambda g, r: (g, r)),
            pl.BlockSpec((DFF, 2 * _BR), lambda g, r: (0, r)),
            pl.BlockSpec((_NP * cg, DFF), lambda g, r: (g, 0)),
            pl.BlockSpec((_NP * cg, _BR), lambda g, r: (g, 0)),
        ],
        out_specs=[
            pl.BlockSpec((cg, 2 * _BR), lambda g, r: (g, r)),
            pl.BlockSpec((1, 1, 2 * _BR), lambda g, r: (g, 0, r)),
        ],
        out_shape=[
            jax.ShapeDtypeStruct((D2, B), jnp.float32),
            jax.ShapeDtypeStruct((_G, 1, B), jnp.float32),
        ],
        scratch_shapes=[
            pltpu.VMEM((_NP * cg, _BR), jnp.float32),
            pltpu.VMEM((_NP * cg, _BR), jnp.float32),
        ],
        compiler_params=pltpu.CompilerParams(
            dimension_semantics=(_ARB, _ARB),
            vmem_limit_bytes=100 * 1024 * 1024,
        ),
    )(x2t, hdnt, w2tp, b2r)

    return yt.T, ldp.sum(axis=(0, 1))
